# Initial kernel scaffold; baseline (speedup 1.0000x reference)
import jax
import jax.numpy as jnp
from jax.experimental import pallas as pl


def kernel(rpn_probs, rpn_bbox, anchors):
    raise NotImplementedError("write your pallas kernel here")


# speedup vs baseline: 2.3476x; 2.3476x over previous
import functools

import jax
import jax.numpy as jnp
from jax.experimental import pallas as pl
from jax.experimental.pallas import tpu as pltpu

_RPN_BBOX_STD_DEV = (0.1, 0.1, 0.2, 0.2)
_PRE_NMS_LIMIT = 6000
_PROPOSAL_COUNT = 1000
_NMS_THRESHOLD = 0.7

_PAD_N = 6144
_ROWS = _PAD_N // 128
_OUT_ROWS = 1008


def _nms_body(sw_ref, a_ref, d_ref, out_ref):
    ay1 = a_ref[0, 0]
    ax1 = a_ref[0, 1]
    ay2 = a_ref[0, 2]
    ax2 = a_ref[0, 3]
    dy = d_ref[0, 0]
    dx = d_ref[0, 1]
    dh = d_ref[0, 2]
    dw = d_ref[0, 3]

    height = ay2 - ay1
    width = ax2 - ax1
    center_y = (ay1 + 0.5 * height) + dy * height
    center_x = (ax1 + 0.5 * width) + dx * width
    height = height * jnp.exp(dh)
    width = width * jnp.exp(dw)
    y1 = center_y - 0.5 * height
    x1 = center_x - 0.5 * width
    y2 = y1 + height
    x2 = x1 + width
    y1 = jnp.clip(y1, 0.0, 1.0)
    x1 = jnp.clip(x1, 0.0, 1.0)
    y2 = jnp.clip(y2, 0.0, 1.0)
    x2 = jnp.clip(x2, 0.0, 1.0)
    areas = (y2 - y1) * (x2 - x1)

    pos = (jax.lax.broadcasted_iota(jnp.int32, (_ROWS, 128), 0) * 128
           + jax.lax.broadcasted_iota(jnp.int32, (_ROWS, 128), 1))
    lane = jax.lax.broadcasted_iota(jnp.int32, (1, 128), 1)
    neg_inf = jnp.float32(-jnp.inf)
    big = jnp.int32(1 << 30)

    def body(i, sw):
        m = jnp.max(sw)
        valid = m > neg_inf
        idx = jnp.min(jnp.where(sw == m, pos, big))
        msk = pos == idx
        by1 = jnp.sum(jnp.where(msk, y1, 0.0))
        bx1 = jnp.sum(jnp.where(msk, x1, 0.0))
        by2 = jnp.sum(jnp.where(msk, y2, 0.0))
        bx2 = jnp.sum(jnp.where(msk, x2, 0.0))
        barea = jnp.sum(jnp.where(msk, areas, 0.0))
        yy1 = jnp.maximum(by1, y1)
        xx1 = jnp.maximum(bx1, x1)
        yy2 = jnp.minimum(by2, y2)
        xx2 = jnp.minimum(bx2, x2)
        inter = jnp.maximum(yy2 - yy1, 0.0) * jnp.maximum(xx2 - xx1, 0.0)
        union = (barea + areas) - inter
        iou = inter / jnp.maximum(union, 1e-8)
        sw = jnp.where((valid & (iou > _NMS_THRESHOLD)) | msk, neg_inf, sw)
        vmask = jnp.where(valid, 1.0, 0.0).astype(jnp.float32)
        row = (jnp.where(lane == 0, by1, 0.0)
               + jnp.where(lane == 1, bx1, 0.0)
               + jnp.where(lane == 2, by2, 0.0)
               + jnp.where(lane == 3, bx2, 0.0)) * vmask
        out_ref[0, pl.ds(i, 1), :] = row
        return sw

    jax.lax.fori_loop(0, _PROPOSAL_COUNT, body, sw_ref[0], unroll=False)


@jax.jit
def kernel(rpn_probs, rpn_bbox, anchors):
    batch, n, _ = anchors.shape
    scores = rpn_probs[:, :, 1]
    std = jnp.asarray(_RPN_BBOX_STD_DEV, jnp.float32).reshape(1, 1, 4)
    deltas = rpn_bbox * std

    top_scores, ix = jax.lax.top_k(scores, _PRE_NMS_LIMIT)
    deltas_g = jnp.take_along_axis(deltas, ix[:, :, None], axis=1)
    anchors_g = jnp.take_along_axis(anchors, ix[:, :, None], axis=1)

    pad = _PAD_N - _PRE_NMS_LIMIT
    sw0 = jnp.concatenate(
        [top_scores, jnp.full((batch, pad), -jnp.inf, jnp.float32)], axis=1
    ).reshape(batch, _ROWS, 128)
    zpad = jnp.zeros((batch, pad, 4), jnp.float32)
    a4 = jnp.concatenate([anchors_g, zpad], axis=1)
    d4 = jnp.concatenate([deltas_g, zpad], axis=1)
    a4 = a4.transpose(0, 2, 1).reshape(batch, 4, _ROWS, 128)
    d4 = d4.transpose(0, 2, 1).reshape(batch, 4, _ROWS, 128)

    out = pl.pallas_call(
        _nms_body,
        grid=(batch,),
        in_specs=[
            pl.BlockSpec((1, _ROWS, 128), lambda b: (b, 0, 0)),
            pl.BlockSpec((1, 4, _ROWS, 128), lambda b: (b, 0, 0, 0)),
            pl.BlockSpec((1, 4, _ROWS, 128), lambda b: (b, 0, 0, 0)),
        ],
        out_specs=pl.BlockSpec((1, _OUT_ROWS, 128), lambda b: (b, 0, 0)),
        out_shape=jax.ShapeDtypeStruct((batch, _OUT_ROWS, 128), jnp.float32),
        compiler_params=pltpu.CompilerParams(
            dimension_semantics=("arbitrary",),
        ),
    )(sw0, a4, d4)
    return out[:, :_PROPOSAL_COUNT, :4]
